# baseline (speedup 1.0000x reference)
import functools

import jax
import jax.numpy as jnp
from jax import lax
from jax.experimental import pallas as pl
from jax.experimental.pallas import tpu as pltpu
from jax.experimental.pallas import tpu_sc as plsc

N = 10000
NP = 10240
E = 320000
D = 128
H0 = 8
F0 = 128
NC = 40
F1 = 48
NEG = 0.2

NCORES = 2
NSUB = 16
NW = NCORES * NSUB
L = 16
BLK = 128
NT = -(-(E // BLK) // NW)
EP = NT * NW * BLK
DUMMY = 10200
RPT = NP // NSUB

_MESH = plsc.VectorSubcoreMesh(core_axis_name="c", subcore_axis_name="s")


def _zero_shared(zv, acc, sid, width):
    zrows = zv.shape[0]
    reps = RPT // zrows

    @pl.loop(0, zrows)
    def _(i):
        @pl.loop(0, width, step=L)
        def _(k):
            zv.at[pl.ds(i, 1), pl.ds(k, L)][...] = jnp.zeros((1, L), jnp.float32)

    @pl.loop(0, reps)
    def _(r):
        pltpu.sync_copy(zv, acc.at[pl.ds(sid * RPT + r * zrows, zrows)])


def _wait(src, dst, sem):
    pltpu.make_async_copy(src, dst, sem).wait()


def _sc_pass_a(src_hbm, dst_hbm, ts_hbm, td_hbm, ex_hbm, dp_hbm,
               si0, di0, ds0, av0, bv0, ev0, si1, di1, ds1, av1, bv1, ev1,
               zv, dacc,
               semi0, semi1, semg0, semg1, semw0, semw1, sems0, sems1):
    c = lax.axis_index("c")
    s = lax.axis_index("s")
    wid = c * NSUB + s
    SI = (si0, si1); DI = (di0, di1); DS = (ds0, ds1)
    AV = (av0, av1); BV = (bv0, bv1); EV = (ev0, ev1)
    SEMI = (semi0, semi1); SEMG = (semg0, semg1)
    SEMW = (semw0, semw1); SEMS = (sems0, sems1)

    _zero_shared(zv, dacc, s, L)
    plsc.subcore_barrier()

    def off_of(t):
        return (wid + NW * t) * BLK

    def issue_idx(t, p):
        off = off_of(t)
        pltpu.async_copy(src_hbm.at[pl.ds(off, BLK)], SI[p], SEMI[p])
        pltpu.async_copy(dst_hbm.at[pl.ds(off, BLK)], DI[p], SEMI[p])

    def wait_idx(p):
        _wait(src_hbm.at[pl.ds(0, BLK)], SI[p], SEMI[p])
        _wait(dst_hbm.at[pl.ds(0, BLK)], DI[p], SEMI[p])

    def issue_gathers(p):
        pltpu.async_copy(ts_hbm.at[SI[p]], AV[p], SEMG[p])
        pltpu.async_copy(td_hbm.at[DI[p]], BV[p], SEMG[p])

    def wait_gathers(p):
        _wait(ts_hbm.at[SI[p]], AV[p], SEMG[p])
        _wait(td_hbm.at[DI[p]], BV[p], SEMG[p])

    def wait_out(p):
        _wait(EV[p], ex_hbm.at[pl.ds(0, BLK)], SEMW[p])
        _wait(EV[p], dacc.at[DS[p]], SEMS[p])

    issue_idx(0, 0)
    issue_idx(1, 1)
    wait_idx(0)
    issue_gathers(0)

    @pl.loop(0, NT // 2 + 1)
    def _(u):
        for p in range(2):
            q = 1 - p
            t = u * 2 + p

            @pl.when(t < NT)
            def _():
                @pl.when(t + 1 < NT)
                def _():
                    wait_idx(q)
                    issue_gathers(q)
                wait_gathers(p)

                @pl.when(t >= 2)
                def _():
                    wait_out(p)

                @pl.when(t + 2 < NT)
                def _():
                    issue_idx(t + 2, p)

                pltpu.sync_copy(dst_hbm.at[pl.ds(off_of(t), BLK)], DS[p])

                @plsc.parallel_loop(0, BLK, unroll=4)
                def _(b):
                    al = (AV[p].at[pl.ds(b, 1), pl.ds(0, L)][...]
                          + BV[p].at[pl.ds(b, 1), pl.ds(0, L)][...])
                    al = jnp.where(al >= 0.0, al, NEG * al)
                    EV[p].at[pl.ds(b, 1), pl.ds(0, L)][...] = jnp.exp(al)

                pltpu.async_copy(EV[p], ex_hbm.at[pl.ds(off_of(t), BLK)], SEMW[p])
                pltpu.async_copy(EV[p], dacc.at[DS[p]], SEMS[p], add=True)

    wait_out(1)
    wait_out(0)

    plsc.subcore_barrier()
    pltpu.sync_copy(dacc.at[pl.ds(s * RPT, RPT)],
                    dp_hbm.at[c, pl.ds(s * RPT, RPT)])


_SC_PARAMS = pltpu.CompilerParams(use_tc_tiling_on_sc=False)

_IDX_T = pltpu.VMEM((BLK,), jnp.int32)
_ROW_T = pltpu.VMEM((BLK, L), jnp.float32)

_pass_a = functools.partial(
    pl.kernel,
    mesh=_MESH,
    compiler_params=_SC_PARAMS,
    out_type=[jax.ShapeDtypeStruct((EP, L), jnp.float32),
              jax.ShapeDtypeStruct((NCORES, NP, L), jnp.float32)],
    scratch_types=[_IDX_T, _IDX_T, _IDX_T, _ROW_T, _ROW_T, _ROW_T,
                   _IDX_T, _IDX_T, _IDX_T, _ROW_T, _ROW_T, _ROW_T,
                   pltpu.VMEM((RPT, L), jnp.float32),
                   pltpu.VMEM_SHARED((NP, L), jnp.float32)]
    + [pltpu.SemaphoreType.DMA] * 8,
)(_sc_pass_a)


def _make_pass_b(fw, nheads):
    chunks_per_head = fw // nheads // L

    def body(src_hbm, dst_hbm, ex_hbm, h_hbm, op_hbm,
             si0, di0, ds0, xv0, hv0, si1, di1, ds1, xv1, hv1,
             zv, oacc,
             semi0, semi1, semx0, semx1, semg0, semg1, sems0, sems1):
        c = lax.axis_index("c")
        s = lax.axis_index("s")
        wid = c * NSUB + s
        SI = (si0, si1); DI = (di0, di1); DS = (ds0, ds1)
        XV = (xv0, xv1); HV = (hv0, hv1)
        SEMI = (semi0, semi1); SEMX = (semx0, semx1)
        SEMG = (semg0, semg1); SEMS = (sems0, sems1)

        _zero_shared(zv, oacc, s, fw)
        plsc.subcore_barrier()

        def off_of(t):
            return (wid + NW * t) * BLK

        def issue_idx(t, p):
            off = off_of(t)
            pltpu.async_copy(src_hbm.at[pl.ds(off, BLK)], SI[p], SEMI[p])
            pltpu.async_copy(dst_hbm.at[pl.ds(off, BLK)], DI[p], SEMI[p])

        def wait_idx(p):
            _wait(src_hbm.at[pl.ds(0, BLK)], SI[p], SEMI[p])
            _wait(dst_hbm.at[pl.ds(0, BLK)], DI[p], SEMI[p])

        def issue_gathers(t, p):
            pltpu.async_copy(ex_hbm.at[pl.ds(off_of(t), BLK)], XV[p], SEMX[p])
            pltpu.async_copy(h_hbm.at[SI[p]], HV[p], SEMG[p])

        def wait_gathers(p):
            _wait(ex_hbm.at[pl.ds(0, BLK)], XV[p], SEMX[p])
            _wait(h_hbm.at[SI[p]], HV[p], SEMG[p])

        def wait_scatter(p):
            _wait(HV[p], oacc.at[DS[p]], SEMS[p])

        issue_idx(0, 0)
        issue_idx(1, 1)
        wait_idx(0)
        issue_gathers(0, 0)

        @pl.loop(0, NT // 2 + 1)
        def _(u):
            for p in range(2):
                q = 1 - p
                t = u * 2 + p

                @pl.when(t < NT)
                def _():
                    @pl.when(t >= 1)
                    def _():
                        wait_scatter(q)

                    @pl.when(t + 1 < NT)
                    def _():
                        wait_idx(q)
                        issue_gathers(t + 1, q)
                    wait_gathers(p)

                    @pl.when(t + 2 < NT)
                    def _():
                        issue_idx(t + 2, p)

                    pltpu.sync_copy(dst_hbm.at[pl.ds(off_of(t), BLK)], DS[p])

                    @plsc.parallel_loop(0, BLK, unroll=8)
                    def _(b):
                        exr = XV[p].at[pl.ds(b, 1), pl.ds(0, L)][...]
                        for hh in range(nheads):
                            asplat = jnp.broadcast_to(exr[0, hh], (1, L))
                            for k in range(chunks_per_head):
                                seg = pl.ds((hh * chunks_per_head + k) * L, L)
                                HV[p].at[pl.ds(b, 1), seg][...] = (
                                    HV[p].at[pl.ds(b, 1), seg][...] * asplat)

                    pltpu.async_copy(HV[p], oacc.at[DS[p]], SEMS[p], add=True)

        wait_scatter(0)

        plsc.subcore_barrier()
        pltpu.sync_copy(oacc.at[pl.ds(s * RPT, RPT)],
                        op_hbm.at[c, pl.ds(s * RPT, RPT)])

    hrow_t = pltpu.VMEM((BLK, fw), jnp.float32)
    return functools.partial(
        pl.kernel,
        mesh=_MESH,
        compiler_params=_SC_PARAMS,
        out_type=[jax.ShapeDtypeStruct((NCORES, NP, fw), jnp.float32)],
        scratch_types=[_IDX_T, _IDX_T, _IDX_T, _ROW_T, hrow_t,
                       _IDX_T, _IDX_T, _IDX_T, _ROW_T, hrow_t,
                       pltpu.VMEM((32, fw), jnp.float32),
                       pltpu.VMEM_SHARED((NP, fw), jnp.float32)]
        + [pltpu.SemaphoreType.DMA] * 8,
    )(body)


_pass_b0 = _make_pass_b(F0, H0)
_pass_b1 = _make_pass_b(F1, 1)


def _tc_transform(x, w, a_s, a_d):
    n, fw = x.shape[0], w.shape[1]

    def body(x_ref, w_ref, as_ref, ad_ref, h_ref, ts_ref, td_ref):
        h = jnp.dot(x_ref[...], w_ref[...], preferred_element_type=jnp.float32)
        h_ref[...] = h
        ts_ref[...] = jnp.dot(h, as_ref[...], preferred_element_type=jnp.float32)
        td_ref[...] = jnp.dot(h, ad_ref[...], preferred_element_type=jnp.float32)

    return pl.pallas_call(
        body,
        out_shape=[jax.ShapeDtypeStruct((n, fw), jnp.float32),
                   jax.ShapeDtypeStruct((n, L), jnp.float32),
                   jax.ShapeDtypeStruct((n, L), jnp.float32)],
    )(x, w, a_s, a_d)


def _tc_dinv(dp):
    def body(dp_ref, o_ref):
        o_ref[...] = 1.0 / (dp_ref[0] + dp_ref[1] + 1e-16)

    return pl.pallas_call(
        body, out_shape=jax.ShapeDtypeStruct((NP, L), jnp.float32))(dp)


RB = 1280


def _tc_mid(p, dinv, b0row, w1p, as1, ad1):
    def body(p_ref, di_ref, b_ref, w_ref, as_ref, ad_ref, h_ref, ts_ref, td_ref):
        dexp = jnp.reshape(
            jnp.broadcast_to(di_ref[...][:, :H0, None], (RB, H0, F0 // H0)),
            (RB, F0))
        sres = (p_ref[0] + p_ref[1]) * dexp + b_ref[...]
        hm = jnp.where(sres > 0.0, sres, jnp.exp(sres) - 1.0)
        h1 = jnp.dot(hm, w_ref[...], preferred_element_type=jnp.float32)
        h_ref[...] = h1
        ts_ref[...] = jnp.dot(h1, as_ref[...], preferred_element_type=jnp.float32)
        td_ref[...] = jnp.dot(h1, ad_ref[...], preferred_element_type=jnp.float32)

    return pl.pallas_call(
        body,
        grid=(NP // RB,),
        in_specs=[pl.BlockSpec((2, RB, F0), lambda i: (0, i, 0)),
                  pl.BlockSpec((RB, L), lambda i: (i, 0)),
                  pl.BlockSpec((1, F0), lambda i: (0, 0)),
                  pl.BlockSpec((F0, F1), lambda i: (0, 0)),
                  pl.BlockSpec((F1, L), lambda i: (0, 0)),
                  pl.BlockSpec((F1, L), lambda i: (0, 0))],
        out_specs=[pl.BlockSpec((RB, F1), lambda i: (i, 0)),
                   pl.BlockSpec((RB, L), lambda i: (i, 0)),
                   pl.BlockSpec((RB, L), lambda i: (i, 0))],
        out_shape=[jax.ShapeDtypeStruct((NP, F1), jnp.float32),
                   jax.ShapeDtypeStruct((NP, L), jnp.float32),
                   jax.ShapeDtypeStruct((NP, L), jnp.float32)],
    )(p, dinv, b0row, w1p, as1, ad1)


def _tc_final(q, dinv, b1row):
    def body(q_ref, di_ref, b_ref, o_ref):
        o_ref[...] = ((q_ref[0] + q_ref[1])
                      * jnp.broadcast_to(di_ref[...][:, :1], (RB, F1))
                      + b_ref[...])

    return pl.pallas_call(
        body,
        grid=(NP // RB,),
        in_specs=[pl.BlockSpec((2, RB, F1), lambda i: (0, i, 0)),
                  pl.BlockSpec((RB, L), lambda i: (i, 0)),
                  pl.BlockSpec((1, F1), lambda i: (0, 0))],
        out_specs=pl.BlockSpec((RB, F1), lambda i: (i, 0)),
        out_shape=jax.ShapeDtypeStruct((NP, F1), jnp.float32))(q, dinv, b1row)


def kernel(x, edge_index, W0, att_src0, att_dst0, b0, W1, att_src1, att_dst1, b1):
    pad = jnp.zeros((EP - E,), jnp.int32)
    src = jnp.concatenate([edge_index[0], pad])
    dst = jnp.concatenate([edge_index[1], pad + DUMMY])

    rows = jnp.arange(F0)
    cols = jnp.repeat(jnp.arange(H0), F0 // H0)
    a_s0 = jnp.zeros((F0, L), jnp.float32).at[rows, cols].set(att_src0.reshape(F0))
    a_d0 = jnp.zeros((F0, L), jnp.float32).at[rows, cols].set(att_dst0.reshape(F0))
    w1p = jnp.zeros((F0, F1), jnp.float32).at[:, :NC].set(W1)
    a_s1 = jnp.zeros((F1, L), jnp.float32).at[:NC, 0].set(att_src1[0])
    a_d1 = jnp.zeros((F1, L), jnp.float32).at[:NC, 0].set(att_dst1[0])
    b0row = b0.reshape(1, F0)
    b1row = jnp.zeros((1, F1), jnp.float32).at[0, :NC].set(b1)

    xp = jnp.zeros((NP, D), jnp.float32).at[:N].set(x)
    h0, ts0, td0 = _tc_transform(xp, W0, a_s0, a_d0)
    ex0, dp0 = _pass_a(src, dst, ts0, td0)
    dinv0 = _tc_dinv(dp0)
    (op0,) = _pass_b0(src, dst, ex0, h0)
    h1, ts1, td1 = _tc_mid(op0, dinv0, b0row, w1p, a_s1, a_d1)
    ex1, dp1 = _pass_a(src, dst, ts1, td1)
    dinv1 = _tc_dinv(dp1)
    (op1,) = _pass_b1(src, dst, ex1, h1)
    out = _tc_final(op1, dinv1, b1row)
    return out[:N, :NC]


# speedup vs baseline: 1.0331x; 1.0331x over previous
import functools

import jax
import jax.numpy as jnp
from jax import lax
from jax.experimental import pallas as pl
from jax.experimental.pallas import tpu as pltpu
from jax.experimental.pallas import tpu_sc as plsc

N = 10000
NP = 10240
E = 320000
D = 128
H0 = 8
F0 = 128
NC = 40
F1 = 48
NEG = 0.2

NCORES = 2
NSUB = 16
NW = NCORES * NSUB
L = 16
BLK = 128
NT = -(-(E // BLK) // NW)
EP = NT * NW * BLK
DUMMY = 10200
RPT = NP // NSUB

_MESH = plsc.VectorSubcoreMesh(core_axis_name="c", subcore_axis_name="s")


def _zero_shared(zv, acc, sid, width):
    zrows = zv.shape[0]
    reps = RPT // zrows

    @pl.loop(0, zrows)
    def _(i):
        @pl.loop(0, width, step=L)
        def _(k):
            zv.at[pl.ds(i, 1), pl.ds(k, L)][...] = jnp.zeros((1, L), jnp.float32)

    @pl.loop(0, reps)
    def _(r):
        pltpu.sync_copy(zv, acc.at[pl.ds(sid * RPT + r * zrows, zrows)])


def _wait(src, dst, sem):
    pltpu.make_async_copy(src, dst, sem).wait()


def _sc_pass_a(src_hbm, dst_hbm, ts_hbm, td_hbm, ex_hbm, dp_hbm,
               si0, di0, ds0, av0, bv0, ev0, si1, di1, ds1, av1, bv1, ev1,
               zv, dacc,
               semi0, semi1, semg0, semg1, semw0, semw1, sems0, sems1):
    c = lax.axis_index("c")
    s = lax.axis_index("s")
    wid = c * NSUB + s
    SI = (si0, si1); DI = (di0, di1); DS = (ds0, ds1)
    AV = (av0, av1); BV = (bv0, bv1); EV = (ev0, ev1)
    SEMI = (semi0, semi1); SEMG = (semg0, semg1)
    SEMW = (semw0, semw1); SEMS = (sems0, sems1)

    _zero_shared(zv, dacc, s, L)
    plsc.subcore_barrier()

    def off_of(t):
        return (wid + NW * t) * BLK

    def issue_idx(t, p):
        off = off_of(t)
        pltpu.async_copy(src_hbm.at[pl.ds(off, BLK)], SI[p], SEMI[p])
        pltpu.async_copy(dst_hbm.at[pl.ds(off, BLK)], DI[p], SEMI[p])

    def wait_idx(p):
        _wait(src_hbm.at[pl.ds(0, BLK)], SI[p], SEMI[p])
        _wait(dst_hbm.at[pl.ds(0, BLK)], DI[p], SEMI[p])

    def issue_gathers(p):
        pltpu.async_copy(ts_hbm.at[SI[p]], AV[p], SEMG[p])
        pltpu.async_copy(td_hbm.at[DI[p]], BV[p], SEMG[p])

    def wait_gathers(p):
        _wait(ts_hbm.at[SI[p]], AV[p], SEMG[p])
        _wait(td_hbm.at[DI[p]], BV[p], SEMG[p])

    def wait_out(p):
        _wait(EV[p], ex_hbm.at[pl.ds(0, BLK)], SEMW[p])
        _wait(EV[p], dacc.at[DS[p]], SEMS[p])

    issue_idx(0, 0)
    issue_idx(1, 1)
    wait_idx(0)
    issue_gathers(0)

    @pl.loop(0, NT // 2 + 1)
    def _(u):
        for p in range(2):
            q = 1 - p
            t = u * 2 + p

            @pl.when(t < NT)
            def _():
                @pl.when(t + 1 < NT)
                def _():
                    wait_idx(q)
                    issue_gathers(q)
                wait_gathers(p)

                @pl.when(t >= 2)
                def _():
                    wait_out(p)

                @pl.when(t + 2 < NT)
                def _():
                    issue_idx(t + 2, p)

                pltpu.sync_copy(dst_hbm.at[pl.ds(off_of(t), BLK)], DS[p])

                @plsc.parallel_loop(0, BLK, unroll=4)
                def _(b):
                    al = (AV[p].at[pl.ds(b, 1), pl.ds(0, L)][...]
                          + BV[p].at[pl.ds(b, 1), pl.ds(0, L)][...])
                    al = jnp.where(al >= 0.0, al, NEG * al)
                    EV[p].at[pl.ds(b, 1), pl.ds(0, L)][...] = jnp.exp(al)

                pltpu.async_copy(EV[p], ex_hbm.at[pl.ds(off_of(t), BLK)], SEMW[p])
                pltpu.async_copy(EV[p], dacc.at[DS[p]], SEMS[p], add=True)

    wait_out(1)
    wait_out(0)

    plsc.subcore_barrier()
    pltpu.sync_copy(dacc.at[pl.ds(s * RPT, RPT)],
                    dp_hbm.at[c, pl.ds(s * RPT, RPT)])


_SC_PARAMS = pltpu.CompilerParams(use_tc_tiling_on_sc=False)

_IDX_T = pltpu.VMEM((BLK,), jnp.int32)
_ROW_T = pltpu.VMEM((BLK, L), jnp.float32)

_pass_a = functools.partial(
    pl.kernel,
    mesh=_MESH,
    compiler_params=_SC_PARAMS,
    out_type=[jax.ShapeDtypeStruct((EP, L), jnp.float32),
              jax.ShapeDtypeStruct((NCORES, NP, L), jnp.float32)],
    scratch_types=[_IDX_T, _IDX_T, _IDX_T, _ROW_T, _ROW_T, _ROW_T,
                   _IDX_T, _IDX_T, _IDX_T, _ROW_T, _ROW_T, _ROW_T,
                   pltpu.VMEM((RPT, L), jnp.float32),
                   pltpu.VMEM_SHARED((NP, L), jnp.float32)]
    + [pltpu.SemaphoreType.DMA] * 8,
)(_sc_pass_a)


def _make_pass_b(fw, nheads):
    chunks_per_head = fw // nheads // L

    def body(src_hbm, dst_hbm, ex_hbm, h_hbm, op_hbm,
             si0, di0, ds0, xv0, hv0, si1, di1, ds1, xv1, hv1,
             zv, oacc,
             semi0, semi1, semx0, semx1, semg0, semg1, sems0, sems1):
        c = lax.axis_index("c")
        s = lax.axis_index("s")
        wid = c * NSUB + s
        SI = (si0, si1); DI = (di0, di1); DS = (ds0, ds1)
        XV = (xv0, xv1); HV = (hv0, hv1)
        SEMI = (semi0, semi1); SEMX = (semx0, semx1)
        SEMG = (semg0, semg1); SEMS = (sems0, sems1)

        _zero_shared(zv, oacc, s, fw)
        plsc.subcore_barrier()

        def off_of(t):
            return (wid + NW * t) * BLK

        def issue_idx(t, p):
            off = off_of(t)
            pltpu.async_copy(src_hbm.at[pl.ds(off, BLK)], SI[p], SEMI[p])
            pltpu.async_copy(dst_hbm.at[pl.ds(off, BLK)], DI[p], SEMI[p])

        def wait_idx(p):
            _wait(src_hbm.at[pl.ds(0, BLK)], SI[p], SEMI[p])
            _wait(dst_hbm.at[pl.ds(0, BLK)], DI[p], SEMI[p])

        def issue_gathers(t, p):
            pltpu.async_copy(ex_hbm.at[pl.ds(off_of(t), BLK)], XV[p], SEMX[p])
            pltpu.async_copy(h_hbm.at[SI[p]], HV[p], SEMG[p])

        def wait_gathers(p):
            _wait(ex_hbm.at[pl.ds(0, BLK)], XV[p], SEMX[p])
            _wait(h_hbm.at[SI[p]], HV[p], SEMG[p])

        def wait_scatter(p):
            _wait(HV[p], oacc.at[DS[p]], SEMS[p])

        issue_idx(0, 0)
        issue_idx(1, 1)
        wait_idx(0)
        issue_gathers(0, 0)

        @pl.loop(0, NT // 2 + 1)
        def _(u):
            for p in range(2):
                q = 1 - p
                t = u * 2 + p

                @pl.when(t < NT)
                def _():
                    @pl.when(t >= 1)
                    def _():
                        wait_scatter(q)

                    @pl.when(t + 1 < NT)
                    def _():
                        wait_idx(q)
                        issue_gathers(t + 1, q)
                    wait_gathers(p)

                    @pl.when(t + 2 < NT)
                    def _():
                        issue_idx(t + 2, p)

                    pltpu.sync_copy(dst_hbm.at[pl.ds(off_of(t), BLK)], DS[p])

                    @plsc.parallel_loop(0, BLK, unroll=4)
                    def _(b):
                        exr = XV[p].at[pl.ds(b, 1), pl.ds(0, L)][...]
                        for hh in range(nheads):
                            asplat = jnp.broadcast_to(exr[0, hh], (1, L))
                            for k in range(chunks_per_head):
                                seg = pl.ds((hh * chunks_per_head + k) * L, L)
                                HV[p].at[pl.ds(b, 1), seg][...] = (
                                    HV[p].at[pl.ds(b, 1), seg][...] * asplat)

                    pltpu.async_copy(HV[p], oacc.at[DS[p]], SEMS[p], add=True)

        wait_scatter(0)

        plsc.subcore_barrier()
        pltpu.sync_copy(oacc.at[pl.ds(s * RPT, RPT)],
                        op_hbm.at[c, pl.ds(s * RPT, RPT)])

    hrow_t = pltpu.VMEM((BLK, fw), jnp.float32)
    return functools.partial(
        pl.kernel,
        mesh=_MESH,
        compiler_params=_SC_PARAMS,
        out_type=[jax.ShapeDtypeStruct((NCORES, NP, fw), jnp.float32)],
        scratch_types=[_IDX_T, _IDX_T, _IDX_T, _ROW_T, hrow_t,
                       _IDX_T, _IDX_T, _IDX_T, _ROW_T, hrow_t,
                       pltpu.VMEM((32, fw), jnp.float32),
                       pltpu.VMEM_SHARED((NP, fw), jnp.float32)]
        + [pltpu.SemaphoreType.DMA] * 8,
    )(body)


_pass_b0 = _make_pass_b(F0, H0)
_pass_b1 = _make_pass_b(F1, 1)


def _tc_transform(x, w, a_s, a_d):
    n, fw = x.shape[0], w.shape[1]

    def body(x_ref, w_ref, as_ref, ad_ref, h_ref, ts_ref, td_ref):
        h = jnp.dot(x_ref[...], w_ref[...], preferred_element_type=jnp.float32)
        h_ref[...] = h
        ts_ref[...] = jnp.dot(h, as_ref[...], preferred_element_type=jnp.float32)
        td_ref[...] = jnp.dot(h, ad_ref[...], preferred_element_type=jnp.float32)

    return pl.pallas_call(
        body,
        out_shape=[jax.ShapeDtypeStruct((n, fw), jnp.float32),
                   jax.ShapeDtypeStruct((n, L), jnp.float32),
                   jax.ShapeDtypeStruct((n, L), jnp.float32)],
    )(x, w, a_s, a_d)


RB = 1280


def _tc_mid(p, dp, b0row, w1p, as1, ad1):
    def body(p_ref, dp_ref, b_ref, w_ref, as_ref, ad_ref, h_ref, ts_ref, td_ref):
        di = 1.0 / (dp_ref[0] + dp_ref[1] + 1e-16)
        dexp = jnp.reshape(
            jnp.broadcast_to(di[:, :H0, None], (RB, H0, F0 // H0)),
            (RB, F0))
        sres = (p_ref[0] + p_ref[1]) * dexp + b_ref[...]
        hm = jnp.where(sres > 0.0, sres, jnp.exp(sres) - 1.0)
        h1 = jnp.dot(hm, w_ref[...], preferred_element_type=jnp.float32)
        h_ref[...] = h1
        ts_ref[...] = jnp.dot(h1, as_ref[...], preferred_element_type=jnp.float32)
        td_ref[...] = jnp.dot(h1, ad_ref[...], preferred_element_type=jnp.float32)

    return pl.pallas_call(
        body,
        grid=(NP // RB,),
        in_specs=[pl.BlockSpec((2, RB, F0), lambda i: (0, i, 0)),
                  pl.BlockSpec((2, RB, L), lambda i: (0, i, 0)),
                  pl.BlockSpec((1, F0), lambda i: (0, 0)),
                  pl.BlockSpec((F0, F1), lambda i: (0, 0)),
                  pl.BlockSpec((F1, L), lambda i: (0, 0)),
                  pl.BlockSpec((F1, L), lambda i: (0, 0))],
        out_specs=[pl.BlockSpec((RB, F1), lambda i: (i, 0)),
                   pl.BlockSpec((RB, L), lambda i: (i, 0)),
                   pl.BlockSpec((RB, L), lambda i: (i, 0))],
        out_shape=[jax.ShapeDtypeStruct((NP, F1), jnp.float32),
                   jax.ShapeDtypeStruct((NP, L), jnp.float32),
                   jax.ShapeDtypeStruct((NP, L), jnp.float32)],
    )(p, dp, b0row, w1p, as1, ad1)


def _tc_final(q, dp, b1row):
    def body(q_ref, dp_ref, b_ref, o_ref):
        di = 1.0 / (dp_ref[0] + dp_ref[1] + 1e-16)
        o_ref[...] = ((q_ref[0] + q_ref[1])
                      * jnp.broadcast_to(di[:, :1], (RB, F1))
                      + b_ref[...])

    return pl.pallas_call(
        body,
        grid=(NP // RB,),
        in_specs=[pl.BlockSpec((2, RB, F1), lambda i: (0, i, 0)),
                  pl.BlockSpec((2, RB, L), lambda i: (0, i, 0)),
                  pl.BlockSpec((1, F1), lambda i: (0, 0))],
        out_specs=pl.BlockSpec((RB, F1), lambda i: (i, 0)),
        out_shape=jax.ShapeDtypeStruct((NP, F1), jnp.float32))(q, dp, b1row)


def kernel(x, edge_index, W0, att_src0, att_dst0, b0, W1, att_src1, att_dst1, b1):
    pad = jnp.zeros((EP - E,), jnp.int32)
    src = jnp.concatenate([edge_index[0], pad])
    dst = jnp.concatenate([edge_index[1], pad + DUMMY])

    rows = jnp.arange(F0)
    cols = jnp.repeat(jnp.arange(H0), F0 // H0)
    a_s0 = jnp.zeros((F0, L), jnp.float32).at[rows, cols].set(att_src0.reshape(F0))
    a_d0 = jnp.zeros((F0, L), jnp.float32).at[rows, cols].set(att_dst0.reshape(F0))
    w1p = jnp.zeros((F0, F1), jnp.float32).at[:, :NC].set(W1)
    a_s1 = jnp.zeros((F1, L), jnp.float32).at[:NC, 0].set(att_src1[0])
    a_d1 = jnp.zeros((F1, L), jnp.float32).at[:NC, 0].set(att_dst1[0])
    b0row = b0.reshape(1, F0)
    b1row = jnp.zeros((1, F1), jnp.float32).at[0, :NC].set(b1)

    xp = jnp.zeros((NP, D), jnp.float32).at[:N].set(x)
    h0, ts0, td0 = _tc_transform(xp, W0, a_s0, a_d0)
    ex0, dp0 = _pass_a(src, dst, ts0, td0)
    (op0,) = _pass_b0(src, dst, ex0, h0)
    h1, ts1, td1 = _tc_mid(op0, dp0, b0row, w1p, a_s1, a_d1)
    ex1, dp1 = _pass_a(src, dst, ts1, td1)
    (op1,) = _pass_b1(src, dst, ex1, h1)
    out = _tc_final(op1, dp1, b1row)
    return out[:N, :NC]
